# Initial kernel scaffold; baseline (speedup 1.0000x reference)
import jax
import jax.numpy as jnp
from jax.experimental import pallas as pl


def kernel(concept_out, encoder_out, boundary_probs, boundary_idx, params):
    raise NotImplementedError("write your pallas kernel here")


# speedup vs baseline: 3874.7800x; 3874.7800x over previous
import functools

import jax
import jax.numpy as jnp
from jax import lax
from jax.experimental import pallas as pl

F32 = jnp.float32
NEG_BIG = -1e30


def _dot_t(a, b):
    return lax.dot_general(a, b, (((1,), (1,)), ((), ())),
                           preferred_element_type=F32)


def _dot(a, b):
    return lax.dot_general(a, b, (((1,), (0,)), ((), ())),
                           preferred_element_type=F32)


def _k1_body(concept_ref, probs_ref, idxcol_ref, idxrow_ref, idxp1_ref,
             down_ref, plug_ref):
    cm = concept_ref[0]
    cd = _dot_t(cm, down_ref[...])
    M = cd.shape[0]
    L = probs_ref.shape[2]

    probs = probs_ref[0]
    idxcol = idxcol_ref[0]
    idxrow = idxrow_ref[0]
    idxp1 = idxp1_ref[0]

    l_iota = lax.broadcasted_iota(jnp.int32, (M, L), 1)
    eq = l_iota == idxcol
    pbc = jnp.sum(jnp.where(eq, jnp.broadcast_to(probs, (M, L)), 0.0),
                  axis=1, keepdims=True)

    lq = jnp.log(1.0 - pbc)
    i0 = lax.broadcasted_iota(jnp.int32, (M, M), 0)
    i1 = lax.broadcasted_iota(jnp.int32, (M, M), 1)
    trilb = i0 >= i1
    trilf = trilb.astype(F32)
    cl_col = _dot(trilf, lq)
    eyef = (i0 == i1).astype(F32)
    cl_row = jnp.sum(eyef * cl_col, axis=0, keepdims=True)
    expo = jnp.where(trilb, cl_col - cl_row, NEG_BIG)
    T = jnp.exp(expo)
    sm = _dot(T, pbc * cd)

    lc = lax.broadcasted_iota(jnp.int32, (L, M), 0)
    mc = lax.broadcasted_iota(jnp.int32, (L, M), 1)
    ge = (lc >= idxrow) | (mc == 0)
    lt = lc < idxp1
    oh = (ge & lt).astype(F32)
    plug_ref[0] = _dot(oh, sm)


def _k2_body(enc_ref, pcol_ref, plug_ref, gw_ref, gb_ref, dw_ref, out_ref):
    enc = enc_ref[0]
    g = jax.nn.sigmoid(_dot_t(enc, gw_ref[...]) + gb_ref[...])
    ge = (1.0 - pcol_ref[0]) * g * enc
    out_ref[0] = _dot_t(ge, dw_ref[...]) + plug_ref[0]


def _k3_body(nlayers, chunk, h0_ref, *refs):
    h = h0_ref[0]
    L, DO = h.shape
    nchunk = L // chunk
    C = chunk
    ic0 = lax.broadcasted_iota(jnp.int32, (C, C), 0)
    ic1 = lax.broadcasted_iota(jnp.int32, (C, C), 1)
    trilC = (ic0 >= ic1).astype(F32)

    for layer in range(nlayers):
        (wconv, cb, wr, br, wi, bi, loga, ow, nw) = refs[layer * 9:(layer + 1) * 9]
        xpad = jnp.concatenate([jnp.zeros((8, DO), F32), h], axis=0)
        acc = jnp.broadcast_to(cb[...], (L, DO))
        for k in range(4):
            acc = acc + xpad[5 + k:5 + k + L, :] * wconv[k:k + 1, :]
        xc = acc
        r = jax.nn.sigmoid(_dot_t(xc, wr[...]) + br[...])
        i = jax.nn.sigmoid(_dot_t(xc, wi[...]) + bi[...])
        lab = jnp.log(jax.nn.sigmoid(loga[...]))
        la = (8.0 * lab) * r
        a = jnp.exp(la)
        u = jnp.sqrt(jnp.clip(1.0 - a * a, 0.0, None)) * (i * xc)
        carry = jnp.zeros((1, DO), F32)
        outs = []
        for c in range(nchunk):
            la_c = la[c * C:(c + 1) * C, :]
            u_c = u[c * C:(c + 1) * C, :]
            cl = _dot(trilC, la_c)
            A = jnp.exp(cl)
            scum = _dot(trilC, u_c / A)
            hc = A * (scum + carry)
            outs.append(hc)
            carry = hc[C - 1:C, :]
        hs = jnp.concatenate(outs, axis=0)
        o = _dot_t(hs, ow[...])
        ms = jnp.mean(o * o, axis=1, keepdims=True)
        h = o * lax.rsqrt(ms + 1e-6) * nw[...]

    upw, normw, out_ref = refs[nlayers * 9], refs[nlayers * 9 + 1], refs[-1]
    up = _dot_t(h, upw[...])
    ms = jnp.mean(up * up, axis=1, keepdims=True)
    out_ref[0] = up * lax.rsqrt(ms + 1e-6) * normw[...]


def kernel(concept_out, encoder_out, boundary_probs, boundary_idx, params):
    B, L, D = encoder_out.shape
    M = concept_out.shape[1]
    DO = params['down_W'].shape[0]

    idx = boundary_idx.astype(jnp.int32)
    idxcol = idx.reshape(B, M, 1)
    idxrow = idx.reshape(B, 1, M)
    idxp1 = jnp.concatenate(
        [idx[:, 1:], jnp.full((B, 1), L, jnp.int32)], axis=1).reshape(B, 1, M)
    probs3 = boundary_probs.reshape(B, 1, L)
    pcol = boundary_probs.reshape(B, L, 1)
    down_w = params['down_W']
    gate_b2 = params['gate_b'].reshape(1, D)
    norm_out2 = params['norm_out_w'].reshape(1, D)

    plug = pl.pallas_call(
        _k1_body,
        grid=(B,),
        in_specs=[
            pl.BlockSpec((1, M, D), lambda b: (b, 0, 0)),
            pl.BlockSpec((1, 1, L), lambda b: (b, 0, 0)),
            pl.BlockSpec((1, M, 1), lambda b: (b, 0, 0)),
            pl.BlockSpec((1, 1, M), lambda b: (b, 0, 0)),
            pl.BlockSpec((1, 1, M), lambda b: (b, 0, 0)),
            pl.BlockSpec((DO, D), lambda b: (0, 0)),
        ],
        out_specs=pl.BlockSpec((1, L, DO), lambda b: (b, 0, 0)),
        out_shape=jax.ShapeDtypeStruct((B, L, DO), F32),
    )(concept_out, probs3, idxcol, idxrow, idxp1, down_w)

    TL = 512
    NT = L // TL
    h0 = pl.pallas_call(
        _k2_body,
        grid=(B, NT),
        in_specs=[
            pl.BlockSpec((1, TL, D), lambda b, t: (b, t, 0)),
            pl.BlockSpec((1, TL, 1), lambda b, t: (b, t, 0)),
            pl.BlockSpec((1, TL, DO), lambda b, t: (b, t, 0)),
            pl.BlockSpec((D, D), lambda b, t: (0, 0)),
            pl.BlockSpec((1, D), lambda b, t: (0, 0)),
            pl.BlockSpec((DO, D), lambda b, t: (0, 0)),
        ],
        out_specs=pl.BlockSpec((1, TL, DO), lambda b, t: (b, t, 0)),
        out_shape=jax.ShapeDtypeStruct((B, L, DO), F32),
    )(encoder_out, pcol, plug, params['gate_W'], gate_b2, down_w)

    layer_args = []
    layer_specs = []
    for lp in params['layers']:
        wconv = lp['conv_w'][:, 0, :].T
        layer_args += [wconv, lp['conv_b'].reshape(1, DO),
                       lp['Wr_W'], lp['Wr_b'].reshape(1, DO),
                       lp['Wi_W'], lp['Wi_b'].reshape(1, DO),
                       lp['log_a'].reshape(1, DO),
                       lp['out_W'], lp['norm_w'].reshape(1, DO)]
        layer_specs += [
            pl.BlockSpec((4, DO), lambda b: (0, 0)),
            pl.BlockSpec((1, DO), lambda b: (0, 0)),
            pl.BlockSpec((DO, DO), lambda b: (0, 0)),
            pl.BlockSpec((1, DO), lambda b: (0, 0)),
            pl.BlockSpec((DO, DO), lambda b: (0, 0)),
            pl.BlockSpec((1, DO), lambda b: (0, 0)),
            pl.BlockSpec((1, DO), lambda b: (0, 0)),
            pl.BlockSpec((DO, DO), lambda b: (0, 0)),
            pl.BlockSpec((1, DO), lambda b: (0, 0)),
        ]
    nlayers = len(params['layers'])
    chunk = 256 if L % 256 == 0 else L
    out = pl.pallas_call(
        functools.partial(_k3_body, nlayers, chunk),
        grid=(B,),
        in_specs=[pl.BlockSpec((1, L, DO), lambda b: (b, 0, 0))] + layer_specs + [
            pl.BlockSpec((D, DO), lambda b: (0, 0)),
            pl.BlockSpec((1, D), lambda b: (0, 0)),
        ],
        out_specs=pl.BlockSpec((1, L, D), lambda b: (b, 0, 0)),
        out_shape=jax.ShapeDtypeStruct((B, L, D), F32),
    )(h0, *layer_args, params['up_W'], norm_out2)
    return out
